# baseline (speedup 1.0000x reference)
import functools

import jax
import jax.numpy as jnp
from jax import lax
from jax.experimental import pallas as pl
from jax.experimental.pallas import tpu as pltpu
from jax.experimental.pallas import tpu_sc as plsc

N = 10000
E = 320000
D = 128
B = 64

NW = 32
CHUNK = 128
NCHUNK = E // CHUNK
SUPC = 2
SUPE = SUPC * CHUNK
EH = E // 2
NCH_H = NCHUNK // 2
NGRP = NCH_H // SUPC
BASE_G = NGRP // NW
EXTRA_G = NGRP - BASE_G * NW
ZROWS = 624
ZTAIL = N - 16 * ZROWS
DW = D // 2

def _wid_groups(c, s):
    wid = s * 2 + c
    ng = jnp.where(wid < EXTRA_G, BASE_G + 1, BASE_G)
    bg = wid * BASE_G + jnp.minimum(wid, EXTRA_G)
    return bg, ng


def _sc_gather_body(xt, dst2, src2, xi_out, xj_out, idx_d, idx_s, rows_d,
                    rows_s, sem_i, sem_g, sem_w):
    c = lax.axis_index("c")
    s = lax.axis_index("s")
    bg, ng = _wid_groups(c, s)

    def body(g, carry):
        off = (bg + g) * SUPE
        ch = (bg + g) * SUPC

        @pl.when(g > 0)
        def _():
            prev = (bg + g - 1) * SUPE
            pltpu.make_async_copy(rows_d, xi_out.at[pl.ds(prev, SUPE)],
                                  sem_w).wait()
            pltpu.make_async_copy(rows_s, xj_out.at[pl.ds(prev, SUPE)],
                                  sem_w).wait()

        ci = pltpu.async_copy(dst2.at[pl.ds(ch, SUPC)], idx_d, sem_i)
        cj = pltpu.async_copy(src2.at[pl.ds(ch, SUPC)], idx_s, sem_i)
        ci.wait()
        cj.wait()
        cps = []
        for j in range(SUPC):
            cps.append(pltpu.async_copy(
                xt.at[idx_d.at[j]], rows_d.at[pl.ds(j * CHUNK, CHUNK)],
                sem_g))
            cps.append(pltpu.async_copy(
                xt.at[idx_s.at[j]], rows_s.at[pl.ds(j * CHUNK, CHUNK)],
                sem_g))
        for cp in cps:
            cp.wait()
        pltpu.async_copy(rows_d, xi_out.at[pl.ds(off, SUPE)], sem_w)
        pltpu.async_copy(rows_s, xj_out.at[pl.ds(off, SUPE)], sem_w)
        return carry

    lax.fori_loop(0, ng, body, 0)
    last = (bg + ng - 1) * SUPE
    pltpu.make_async_copy(rows_d, xi_out.at[pl.ds(last, SUPE)], sem_w).wait()
    pltpu.make_async_copy(rows_s, xj_out.at[pl.ds(last, SUPE)], sem_w).wait()


def _sc_scatter_body(m, dst2, zeros, out, idx_v, rows_v, acc_sh, sem_i):
    c = lax.axis_index("c")
    s = lax.axis_index("s")
    bg, ng = _wid_groups(c, s)
    rb = s * ZROWS
    pltpu.sync_copy(zeros.at[pl.ds(rb, ZROWS)], acc_sh.at[pl.ds(rb, ZROWS)])

    @pl.when(s == 15)
    def _():
        pltpu.sync_copy(zeros.at[pl.ds(16 * ZROWS, ZTAIL)],
                        acc_sh.at[pl.ds(16 * ZROWS, ZTAIL)])

    plsc.subcore_barrier()

    def body(g, carry):
        off = (bg + g) * SUPE
        ch = (bg + g) * SUPC
        ci = pltpu.async_copy(dst2.at[pl.ds(ch, SUPC)], idx_v, sem_i)
        cr = pltpu.async_copy(m.at[pl.ds(off, SUPE)], rows_v, sem_i)
        ci.wait()
        cr.wait()
        for j in range(SUPC):
            pltpu.sync_copy(rows_v.at[pl.ds(j * CHUNK, CHUNK)],
                            acc_sh.at[idx_v.at[j]], add=True)
        return carry

    lax.fori_loop(0, ng, body, 0)
    plsc.subcore_barrier()
    pltpu.sync_copy(acc_sh.at[pl.ds(rb, ZROWS)],
                    out.at[pl.ds(c * N + rb, ZROWS)])

    @pl.when(s == 15)
    def _():
        pltpu.sync_copy(acc_sh.at[pl.ds(16 * ZROWS, ZTAIL)],
                        out.at[pl.ds(c * N + 16 * ZROWS, ZTAIL)])


@functools.cache
def _sc_kernels():
    mesh = plsc.VectorSubcoreMesh(core_axis_name="c", subcore_axis_name="s")
    gather = pl.kernel(
        _sc_gather_body, mesh=mesh,
        compiler_params=pltpu.CompilerParams(use_tc_tiling_on_sc=False),
        out_type=[jax.ShapeDtypeStruct((EH, DW), jnp.int32),
                  jax.ShapeDtypeStruct((EH, DW), jnp.int32)],
        scratch_types=[pltpu.VMEM((SUPC, CHUNK), jnp.int32),
                       pltpu.VMEM((SUPC, CHUNK), jnp.int32),
                       pltpu.VMEM((SUPE, DW), jnp.int32),
                       pltpu.VMEM((SUPE, DW), jnp.int32),
                       pltpu.SemaphoreType.DMA,
                       pltpu.SemaphoreType.DMA,
                       pltpu.SemaphoreType.DMA])
    scatter = pl.kernel(
        _sc_scatter_body, mesh=mesh,
        out_type=jax.ShapeDtypeStruct((2 * N, D), jnp.float32),
        scratch_types=[pltpu.VMEM((SUPC, CHUNK), jnp.int32),
                       pltpu.VMEM((SUPE, D), jnp.float32),
                       pltpu.VMEM_SHARED((N, D), jnp.float32),
                       pltpu.SemaphoreType.DMA])
    return gather, scatter


NBLK = 5
NB = N // NBLK
EBLK = 1280
NEB = EH // EBLK


def _embed_body(at_ref, emb_ref, nw_ref, nb_ref, x_ref, xt_ref):
    at = at_ref[...]
    col = lax.broadcasted_iota(jnp.int32, (NB, D), 1)
    oh = jnp.where(at == col, 1.0, 0.0).astype(jnp.float32)
    x = jnp.dot(oh, emb_ref[...], preferred_element_type=jnp.float32)
    x_ref[...] = x
    xt_ref[...] = (jnp.dot(x, nw_ref[...], preferred_element_type=jnp.float32)
                   + nb_ref[...]).astype(jnp.bfloat16)


def _embed(at2d, emb_pad, nw, nb):
    return pl.pallas_call(
        _embed_body,
        grid=(NBLK,),
        in_specs=[pl.BlockSpec((NB, 1), lambda i: (i, 0)),
                  pl.BlockSpec((D, D), lambda i: (0, 0)),
                  pl.BlockSpec((D, D), lambda i: (0, 0)),
                  pl.BlockSpec((1, D), lambda i: (0, 0))],
        out_specs=[pl.BlockSpec((NB, D), lambda i: (i, 0)),
                   pl.BlockSpec((NB, D), lambda i: (i, 0))],
        out_shape=[jax.ShapeDtypeStruct((N, D), jnp.float32),
                   jax.ShapeDtypeStruct((N, D), jnp.bfloat16)],
    )(at2d, emb_pad, nw, nb)


def _msg_body(xi_ref, xj_ref, dist_ref, w1a_ref, w1b_ref, b1_ref, w2_ref,
              b2_ref, erow_ref, eb_ref, m_ref):
    bf = jnp.bfloat16
    h = (jnp.dot(xi_ref[...], w1a_ref[...].astype(bf),
                 preferred_element_type=jnp.float32)
         + jnp.dot(xj_ref[...], w1b_ref[...].astype(bf),
                   preferred_element_type=jnp.float32)
         + b1_ref[...])
    h = jnp.maximum(h, 0.0)
    m = (jnp.dot(h.astype(bf), w2_ref[...].astype(bf),
                 preferred_element_type=jnp.float32)
         + b2_ref[...])
    et = dist_ref[...] * erow_ref[...] + eb_ref[...]
    m_ref[...] = m * et


def _msg(xi, xj, dist2d, w1a, w1b, b1, w2, b2, erow, eb):
    full = lambda i: (0, 0)
    blk = lambda i: (i, 0)
    return pl.pallas_call(
        _msg_body,
        grid=(NEB,),
        in_specs=[pl.BlockSpec((EBLK, D), blk),
                  pl.BlockSpec((EBLK, D), blk),
                  pl.BlockSpec((EBLK, 1), blk),
                  pl.BlockSpec((D, D), full),
                  pl.BlockSpec((D, D), full),
                  pl.BlockSpec((1, D), full),
                  pl.BlockSpec((D, D), full),
                  pl.BlockSpec((1, D), full),
                  pl.BlockSpec((1, D), full),
                  pl.BlockSpec((1, D), full)],
        out_specs=pl.BlockSpec((EBLK, D), blk),
        out_shape=jax.ShapeDtypeStruct((EH, D), jnp.float32),
    )(xi, xj, dist2d, w1a, w1b, b1, w2, b2, erow, eb)


def _softplus(u):
    return jnp.maximum(u, 0.0) + jnp.log(1.0 + jnp.exp(-jnp.abs(u)))


def _update_body(a0_ref, a1_ref, b0_ref, b1_ref, x_ref, w1a_ref, w1b_ref,
                 b1w_ref, w2_ref, b2_ref, g_ref, bln_ref, nw_ref, nb_ref,
                 xn_ref, xt_ref, residual):
    aggr = (a0_ref[...] + a1_ref[...]) + (b0_ref[...] + b1_ref[...])
    x = x_ref[...]
    u = (jnp.dot(aggr, w1a_ref[...], preferred_element_type=jnp.float32)
         + jnp.dot(x, w1b_ref[...], preferred_element_type=jnp.float32)
         + b1w_ref[...])
    u = jnp.maximum(u, 0.0)
    u = (jnp.dot(u, w2_ref[...], preferred_element_type=jnp.float32)
         + b2_ref[...])
    u = _softplus(u)
    mu = jnp.mean(u, axis=-1, keepdims=True)
    d = u - mu
    var = jnp.mean(d * d, axis=-1, keepdims=True)
    xnew = d * lax.rsqrt(var + 1e-5) * g_ref[...] + bln_ref[...]
    if residual:
        xnew = x + xnew
    xn_ref[...] = xnew
    xt_ref[...] = (jnp.dot(xnew, nw_ref[...],
                           preferred_element_type=jnp.float32)
                   + nb_ref[...]).astype(jnp.bfloat16)


def _update(aggrA, aggrB, x, w1a, w1b, b1, w2, b2, g, bln, nw, nb, residual):
    full = lambda i: (0, 0)
    blk = lambda i: (i, 0)
    blk2 = lambda i: (i + NBLK, 0)
    return pl.pallas_call(
        functools.partial(_update_body, residual=residual),
        grid=(NBLK,),
        in_specs=[pl.BlockSpec((NB, D), blk),
                  pl.BlockSpec((NB, D), blk2),
                  pl.BlockSpec((NB, D), blk),
                  pl.BlockSpec((NB, D), blk2),
                  pl.BlockSpec((NB, D), blk),
                  pl.BlockSpec((D, D), full),
                  pl.BlockSpec((D, D), full),
                  pl.BlockSpec((1, D), full),
                  pl.BlockSpec((D, D), full),
                  pl.BlockSpec((1, D), full),
                  pl.BlockSpec((1, D), full),
                  pl.BlockSpec((1, D), full),
                  pl.BlockSpec((D, D), full),
                  pl.BlockSpec((1, D), full)],
        out_specs=[pl.BlockSpec((NB, D), blk),
                   pl.BlockSpec((NB, D), blk)],
        out_shape=[jax.ShapeDtypeStruct((N, D), jnp.float32),
                   jax.ShapeDtypeStruct((N, D), jnp.bfloat16)],
    )(aggrA, aggrA, aggrB, aggrB, x, w1a, w1b, b1, w2, b2, g, bln, nw, nb)


def _pool_body(x_ref, batch_ref, w1_ref, b1_ref, w2_ref, b2_ref, w3_ref,
               b3_ref, out_ref):
    bt = batch_ref[...]
    col = lax.broadcasted_iota(jnp.int32, (N, B), 1)
    oh = jnp.where(bt == col, 1.0, 0.0).astype(jnp.float32)
    dn = (((0,), (0,)), ((), ()))
    sums = lax.dot_general(oh, x_ref[...], dn,
                           preferred_element_type=jnp.float32)
    ones = jnp.ones((N, 1), jnp.float32)
    counts = lax.dot_general(oh, ones, dn,
                             preferred_element_type=jnp.float32)
    pooled = sums / jnp.maximum(counts, 1.0)
    h = jnp.maximum(jnp.dot(pooled, w1_ref[...],
                            preferred_element_type=jnp.float32)
                    + b1_ref[...], 0.0)
    h = jnp.maximum(jnp.dot(h, w2_ref[...],
                            preferred_element_type=jnp.float32)
                    + b2_ref[...], 0.0)
    out_ref[...] = _softplus(jnp.dot(h, w3_ref[...],
                                     preferred_element_type=jnp.float32)
                             + b3_ref[...])


def _pool(x, batch2d, w1, b1, w2, b2, w3, b3):
    return pl.pallas_call(
        _pool_body,
        out_shape=jax.ShapeDtypeStruct((B, 1), jnp.float32),
    )(x, batch2d, w1, b1, w2, b2, w3, b3)


def kernel(atom_types, edge_index, distances, batch, emb, node_W, node_b,
           edge_W, edge_b, msg_W1, msg_b1, msg_W2, msg_b2, upd_W1, upd_b1,
           upd_W2, upd_b2, ln_g, ln_b, p_W1, p_b1, p_W2, p_b2, p_W3, p_b3):
    at2d = atom_types.astype(jnp.int32).reshape(N, 1)
    batch2d = batch.astype(jnp.int32).reshape(N, 1)
    src2 = edge_index[0].astype(jnp.int32).reshape(NCHUNK, CHUNK)
    dst2 = edge_index[1].astype(jnp.int32).reshape(NCHUNK, CHUNK)
    dist2d = distances.reshape(E, 1)
    halves = [(dst2[:NCH_H], src2[:NCH_H], dist2d[:EH]),
              (dst2[NCH_H:], src2[NCH_H:], dist2d[EH:])]
    emb_pad = jnp.zeros((D, D), jnp.float32).at[:emb.shape[0]].set(emb)
    zeros_nd = jnp.zeros((N, D), jnp.float32)

    def pack(xb):
        return lax.bitcast_convert_type(xb.reshape(N, DW, 2), jnp.int32)

    def unpack(a):
        return lax.bitcast_convert_type(a, jnp.bfloat16).reshape(EH, D)

    sc_gather, sc_scatter = _sc_kernels()
    x, xt = _embed(at2d, emb_pad, node_W[0], node_b[0].reshape(1, D))
    for l in range(2):
        xt32 = pack(xt)
        aggs = []
        for dsth, srch, disth in halves:
            xi, xj = sc_gather(xt32, dsth, srch)
            m = _msg(unpack(xi), unpack(xj), disth,
                     msg_W1[l][:D], msg_W1[l][D:], msg_b1[l].reshape(1, D),
                     msg_W2[l], msg_b2[l].reshape(1, D),
                     edge_W[l], edge_b[l].reshape(1, D))
            aggs.append(sc_scatter(m, dsth, zeros_nd))
        nl = min(l + 1, 1)
        x, xt = _update(aggs[0], aggs[1], x,
                        upd_W1[l][:D], upd_W1[l][D:], upd_b1[l].reshape(1, D),
                        upd_W2[l], upd_b2[l].reshape(1, D),
                        ln_g[l].reshape(1, D), ln_b[l].reshape(1, D),
                        node_W[nl], node_b[nl].reshape(1, D),
                        residual=(l > 0))
    return _pool(x, batch2d, p_W1, p_b1.reshape(1, D // 2),
                 p_W2, p_b2.reshape(1, D // 4), p_W3, p_b3.reshape(1, 1))


# speedup vs baseline: 3.0245x; 3.0245x over previous
import functools

import jax
import jax.numpy as jnp
from jax import lax
from jax.experimental import pallas as pl
from jax.experimental.pallas import tpu as pltpu
from jax.experimental.pallas import tpu_sc as plsc

N = 10000
E = 320000
D = 128
B = 64

NW = 32
CHUNK = 128
NCHUNK = E // CHUNK
SUPC = 2
SUPE = SUPC * CHUNK
EH = E // 2
NCH_H = NCHUNK // 2
NGRP = NCH_H // SUPC
BASE_G = NGRP // NW
EXTRA_G = NGRP - BASE_G * NW
ZROWS = 624
ZTAIL = N - 16 * ZROWS

def _wid_groups(c, s):
    wid = s * 2 + c
    ng = jnp.where(wid < EXTRA_G, BASE_G + 1, BASE_G)
    bg = wid * BASE_G + jnp.minimum(wid, EXTRA_G)
    return bg, ng


def _sc_gather_body(xt, dst2, src2, xi_out, xj_out, idx_d, idx_s, rows_d,
                    rows_s, sem_i, sem_g, sem_w):
    c = lax.axis_index("c")
    s = lax.axis_index("s")
    bg, ng = _wid_groups(c, s)

    def body(g, carry):
        off = (bg + g) * SUPE
        ch = (bg + g) * SUPC

        @pl.when(g > 0)
        def _():
            prev = (bg + g - 1) * SUPE
            pltpu.make_async_copy(rows_d, xi_out.at[pl.ds(prev, SUPE)],
                                  sem_w).wait()
            pltpu.make_async_copy(rows_s, xj_out.at[pl.ds(prev, SUPE)],
                                  sem_w).wait()

        ci = pltpu.async_copy(dst2.at[pl.ds(ch, SUPC)], idx_d, sem_i)
        cj = pltpu.async_copy(src2.at[pl.ds(ch, SUPC)], idx_s, sem_i)
        ci.wait()
        cj.wait()
        cps = []
        for j in range(SUPC):
            cps.append(pltpu.async_copy(
                xt.at[idx_d.at[j]], rows_d.at[pl.ds(j * CHUNK, CHUNK)],
                sem_g))
            cps.append(pltpu.async_copy(
                xt.at[idx_s.at[j]], rows_s.at[pl.ds(j * CHUNK, CHUNK)],
                sem_g))
        for cp in cps:
            cp.wait()
        pltpu.async_copy(rows_d, xi_out.at[pl.ds(off, SUPE)], sem_w)
        pltpu.async_copy(rows_s, xj_out.at[pl.ds(off, SUPE)], sem_w)
        return carry

    lax.fori_loop(0, ng, body, 0)
    last = (bg + ng - 1) * SUPE
    pltpu.make_async_copy(rows_d, xi_out.at[pl.ds(last, SUPE)], sem_w).wait()
    pltpu.make_async_copy(rows_s, xj_out.at[pl.ds(last, SUPE)], sem_w).wait()


def _sc_scatter_body(m, dst2, zeros, out, idx_v, rows_v, acc_sh, sem_i,
                     sem_a):
    c = lax.axis_index("c")
    s = lax.axis_index("s")
    bg, ng = _wid_groups(c, s)
    rb = s * ZROWS
    pltpu.sync_copy(zeros.at[pl.ds(rb, ZROWS)], acc_sh.at[pl.ds(rb, ZROWS)])

    @pl.when(s == 15)
    def _():
        pltpu.sync_copy(zeros.at[pl.ds(16 * ZROWS, ZTAIL)],
                        acc_sh.at[pl.ds(16 * ZROWS, ZTAIL)])

    plsc.subcore_barrier()

    def body(g, carry):
        off = (bg + g) * SUPE
        ch = (bg + g) * SUPC
        ci = pltpu.async_copy(dst2.at[pl.ds(ch, SUPC)], idx_v, sem_i)
        cr = pltpu.async_copy(m.at[pl.ds(off, SUPE)], rows_v, sem_i)
        ci.wait()
        cr.wait()
        adds = [pltpu.async_copy(rows_v.at[pl.ds(j * CHUNK, CHUNK)],
                                 acc_sh.at[idx_v.at[j]], sem_a, add=True)
                for j in range(SUPC)]
        for a in adds:
            a.wait()
        return carry

    lax.fori_loop(0, ng, body, 0)
    plsc.subcore_barrier()
    pltpu.sync_copy(acc_sh.at[pl.ds(rb, ZROWS)],
                    out.at[pl.ds(c * N + rb, ZROWS)])

    @pl.when(s == 15)
    def _():
        pltpu.sync_copy(acc_sh.at[pl.ds(16 * ZROWS, ZTAIL)],
                        out.at[pl.ds(c * N + 16 * ZROWS, ZTAIL)])


@functools.cache
def _sc_kernels():
    mesh = plsc.VectorSubcoreMesh(core_axis_name="c", subcore_axis_name="s")
    gather = pl.kernel(
        _sc_gather_body, mesh=mesh,
        out_type=[jax.ShapeDtypeStruct((EH, D), jnp.float32),
                  jax.ShapeDtypeStruct((EH, D), jnp.float32)],
        scratch_types=[pltpu.VMEM((SUPC, CHUNK), jnp.int32),
                       pltpu.VMEM((SUPC, CHUNK), jnp.int32),
                       pltpu.VMEM((SUPE, D), jnp.float32),
                       pltpu.VMEM((SUPE, D), jnp.float32),
                       pltpu.SemaphoreType.DMA,
                       pltpu.SemaphoreType.DMA,
                       pltpu.SemaphoreType.DMA])
    scatter = pl.kernel(
        _sc_scatter_body, mesh=mesh,
        out_type=jax.ShapeDtypeStruct((2 * N, D), jnp.float32),
        scratch_types=[pltpu.VMEM((SUPC, CHUNK), jnp.int32),
                       pltpu.VMEM((SUPE, D), jnp.float32),
                       pltpu.VMEM_SHARED((N, D), jnp.float32),
                       pltpu.SemaphoreType.DMA,
                       pltpu.SemaphoreType.DMA])
    return gather, scatter


NBLK = 5
NB = N // NBLK
EBLK = 1280
NEB = EH // EBLK


def _embed_body(at_ref, emb_ref, nw_ref, nb_ref, x_ref, xt_ref):
    at = at_ref[...]
    col = lax.broadcasted_iota(jnp.int32, (NB, D), 1)
    oh = jnp.where(at == col, 1.0, 0.0).astype(jnp.float32)
    x = jnp.dot(oh, emb_ref[...], preferred_element_type=jnp.float32)
    x_ref[...] = x
    xt_ref[...] = (jnp.dot(x, nw_ref[...], preferred_element_type=jnp.float32)
                   + nb_ref[...])


def _embed(at2d, emb_pad, nw, nb):
    return pl.pallas_call(
        _embed_body,
        grid=(NBLK,),
        in_specs=[pl.BlockSpec((NB, 1), lambda i: (i, 0)),
                  pl.BlockSpec((D, D), lambda i: (0, 0)),
                  pl.BlockSpec((D, D), lambda i: (0, 0)),
                  pl.BlockSpec((1, D), lambda i: (0, 0))],
        out_specs=[pl.BlockSpec((NB, D), lambda i: (i, 0)),
                   pl.BlockSpec((NB, D), lambda i: (i, 0))],
        out_shape=[jax.ShapeDtypeStruct((N, D), jnp.float32),
                   jax.ShapeDtypeStruct((N, D), jnp.float32)],
    )(at2d, emb_pad, nw, nb)


def _msg_body(xi_ref, xj_ref, dist_ref, w1a_ref, w1b_ref, b1_ref, w2_ref,
              b2_ref, erow_ref, eb_ref, m_ref):
    bf = jnp.bfloat16
    h = (jnp.dot(xi_ref[...].astype(bf), w1a_ref[...].astype(bf),
                 preferred_element_type=jnp.float32)
         + jnp.dot(xj_ref[...].astype(bf), w1b_ref[...].astype(bf),
                   preferred_element_type=jnp.float32)
         + b1_ref[...])
    h = jnp.maximum(h, 0.0)
    m = (jnp.dot(h.astype(bf), w2_ref[...].astype(bf),
                 preferred_element_type=jnp.float32)
         + b2_ref[...])
    et = dist_ref[...] * erow_ref[...] + eb_ref[...]
    m_ref[...] = m * et


def _msg(xi, xj, dist2d, w1a, w1b, b1, w2, b2, erow, eb):
    full = lambda i: (0, 0)
    blk = lambda i: (i, 0)
    return pl.pallas_call(
        _msg_body,
        grid=(NEB,),
        in_specs=[pl.BlockSpec((EBLK, D), blk),
                  pl.BlockSpec((EBLK, D), blk),
                  pl.BlockSpec((EBLK, 1), blk),
                  pl.BlockSpec((D, D), full),
                  pl.BlockSpec((D, D), full),
                  pl.BlockSpec((1, D), full),
                  pl.BlockSpec((D, D), full),
                  pl.BlockSpec((1, D), full),
                  pl.BlockSpec((1, D), full),
                  pl.BlockSpec((1, D), full)],
        out_specs=pl.BlockSpec((EBLK, D), blk),
        out_shape=jax.ShapeDtypeStruct((EH, D), jnp.float32),
    )(xi, xj, dist2d, w1a, w1b, b1, w2, b2, erow, eb)


def _softplus(u):
    return jnp.maximum(u, 0.0) + jnp.log(1.0 + jnp.exp(-jnp.abs(u)))


def _update_body(a0_ref, a1_ref, b0_ref, b1_ref, x_ref, w1a_ref, w1b_ref,
                 b1w_ref, w2_ref, b2_ref, g_ref, bln_ref, nw_ref, nb_ref,
                 xn_ref, xt_ref, residual):
    aggr = (a0_ref[...] + a1_ref[...]) + (b0_ref[...] + b1_ref[...])
    x = x_ref[...]
    u = (jnp.dot(aggr, w1a_ref[...], preferred_element_type=jnp.float32)
         + jnp.dot(x, w1b_ref[...], preferred_element_type=jnp.float32)
         + b1w_ref[...])
    u = jnp.maximum(u, 0.0)
    u = (jnp.dot(u, w2_ref[...], preferred_element_type=jnp.float32)
         + b2_ref[...])
    u = _softplus(u)
    mu = jnp.mean(u, axis=-1, keepdims=True)
    d = u - mu
    var = jnp.mean(d * d, axis=-1, keepdims=True)
    xnew = d * lax.rsqrt(var + 1e-5) * g_ref[...] + bln_ref[...]
    if residual:
        xnew = x + xnew
    xn_ref[...] = xnew
    xt_ref[...] = (jnp.dot(xnew, nw_ref[...],
                           preferred_element_type=jnp.float32) + nb_ref[...])


def _update(aggrA, aggrB, x, w1a, w1b, b1, w2, b2, g, bln, nw, nb, residual):
    full = lambda i: (0, 0)
    blk = lambda i: (i, 0)
    blk2 = lambda i: (i + NBLK, 0)
    return pl.pallas_call(
        functools.partial(_update_body, residual=residual),
        grid=(NBLK,),
        in_specs=[pl.BlockSpec((NB, D), blk),
                  pl.BlockSpec((NB, D), blk2),
                  pl.BlockSpec((NB, D), blk),
                  pl.BlockSpec((NB, D), blk2),
                  pl.BlockSpec((NB, D), blk),
                  pl.BlockSpec((D, D), full),
                  pl.BlockSpec((D, D), full),
                  pl.BlockSpec((1, D), full),
                  pl.BlockSpec((D, D), full),
                  pl.BlockSpec((1, D), full),
                  pl.BlockSpec((1, D), full),
                  pl.BlockSpec((1, D), full),
                  pl.BlockSpec((D, D), full),
                  pl.BlockSpec((1, D), full)],
        out_specs=[pl.BlockSpec((NB, D), blk),
                   pl.BlockSpec((NB, D), blk)],
        out_shape=[jax.ShapeDtypeStruct((N, D), jnp.float32),
                   jax.ShapeDtypeStruct((N, D), jnp.float32)],
    )(aggrA, aggrA, aggrB, aggrB, x, w1a, w1b, b1, w2, b2, g, bln, nw, nb)


def _pool_body(x_ref, batch_ref, w1_ref, b1_ref, w2_ref, b2_ref, w3_ref,
               b3_ref, out_ref):
    bt = batch_ref[...]
    col = lax.broadcasted_iota(jnp.int32, (N, B), 1)
    oh = jnp.where(bt == col, 1.0, 0.0).astype(jnp.float32)
    dn = (((0,), (0,)), ((), ()))
    sums = lax.dot_general(oh, x_ref[...], dn,
                           preferred_element_type=jnp.float32)
    ones = jnp.ones((N, 1), jnp.float32)
    counts = lax.dot_general(oh, ones, dn,
                             preferred_element_type=jnp.float32)
    pooled = sums / jnp.maximum(counts, 1.0)
    h = jnp.maximum(jnp.dot(pooled, w1_ref[...],
                            preferred_element_type=jnp.float32)
                    + b1_ref[...], 0.0)
    h = jnp.maximum(jnp.dot(h, w2_ref[...],
                            preferred_element_type=jnp.float32)
                    + b2_ref[...], 0.0)
    out_ref[...] = _softplus(jnp.dot(h, w3_ref[...],
                                     preferred_element_type=jnp.float32)
                             + b3_ref[...])


def _pool(x, batch2d, w1, b1, w2, b2, w3, b3):
    return pl.pallas_call(
        _pool_body,
        out_shape=jax.ShapeDtypeStruct((B, 1), jnp.float32),
    )(x, batch2d, w1, b1, w2, b2, w3, b3)


def kernel(atom_types, edge_index, distances, batch, emb, node_W, node_b,
           edge_W, edge_b, msg_W1, msg_b1, msg_W2, msg_b2, upd_W1, upd_b1,
           upd_W2, upd_b2, ln_g, ln_b, p_W1, p_b1, p_W2, p_b2, p_W3, p_b3):
    at2d = atom_types.astype(jnp.int32).reshape(N, 1)
    batch2d = batch.astype(jnp.int32).reshape(N, 1)
    src2 = edge_index[0].astype(jnp.int32).reshape(NCHUNK, CHUNK)
    dst2 = edge_index[1].astype(jnp.int32).reshape(NCHUNK, CHUNK)
    dist2d = distances.reshape(E, 1)
    halves = [(dst2[:NCH_H], src2[:NCH_H], dist2d[:EH]),
              (dst2[NCH_H:], src2[NCH_H:], dist2d[EH:])]
    emb_pad = jnp.zeros((D, D), jnp.float32).at[:emb.shape[0]].set(emb)
    zeros_nd = jnp.zeros((N, D), jnp.float32)

    sc_gather, sc_scatter = _sc_kernels()
    x, xt = _embed(at2d, emb_pad, node_W[0], node_b[0].reshape(1, D))
    for l in range(2):
        aggs = []
        for dsth, srch, disth in halves:
            xi, xj = sc_gather(xt, dsth, srch)
            m = _msg(xi, xj, disth,
                     msg_W1[l][:D], msg_W1[l][D:], msg_b1[l].reshape(1, D),
                     msg_W2[l], msg_b2[l].reshape(1, D),
                     edge_W[l], edge_b[l].reshape(1, D))
            aggs.append(sc_scatter(m, dsth, zeros_nd))
        nl = min(l + 1, 1)
        x, xt = _update(aggs[0], aggs[1], x,
                        upd_W1[l][:D], upd_W1[l][D:], upd_b1[l].reshape(1, D),
                        upd_W2[l], upd_b2[l].reshape(1, D),
                        ln_g[l].reshape(1, D), ln_b[l].reshape(1, D),
                        node_W[nl], node_b[nl].reshape(1, D),
                        residual=(l > 0))
    return _pool(x, batch2d, p_W1, p_b1.reshape(1, D // 2),
                 p_W2, p_b2.reshape(1, D // 4), p_W3, p_b3.reshape(1, 1))
